# baseline (speedup 1.0000x reference)
import functools

import jax
import jax.numpy as jnp
from jax import lax
from jax.experimental import pallas as pl
from jax.experimental.pallas import tpu as pltpu
from jax.experimental.pallas import tpu_sc as plsc

N = 10000
E = 320000
F_IN = 128
F_OUT = 256
PERIODS = 12
HID = 128
OUT_DIM = 1

NCORES = 2
NTILES = 16
EBLK = 64
BPT = 314
E_PAD = NTILES * BPT * EBLK
FC = 64
NHALF = F_IN // FC
CHUNKS = PERIODS * NHALF
C_PER_CORE = CHUNKS // NCORES
N_PAD = 10240
ROWS = N_PAD // NTILES
ZROWS = 128
DEGW = 16


def _sc_body(src_hbm, dst_hbm, xf_hbm, zeros_hbm, ones_hbm, zcol_hbm,
             agg_out, deg_out,
             sidx, didx, rows_a, rows_b, zbuf, ones, accum, dacc,
             gsem_a, gsem_b):
    c = lax.axis_index("c")
    s = lax.axis_index("s")
    row0 = s * ROWS
    bufs = (rows_a, rows_b)
    gsems = (gsem_a, gsem_b)

    pltpu.sync_copy(zeros_hbm, zbuf)
    pltpu.sync_copy(ones_hbm, ones)
    pltpu.sync_copy(dst_hbm.at[s], didx)
    for z in range(ROWS // ZROWS):
        pltpu.sync_copy(zbuf, accum.at[pl.ds(row0 + z * ZROWS, ZROWS)])

    @pl.when(c == 0)
    def _():
        pltpu.sync_copy(zcol_hbm, dacc.at[pl.ds(row0, ROWS)])

    plsc.subcore_barrier()

    for ci in range(C_PER_CORE):
        q = c * C_PER_CORE + ci
        pltpu.sync_copy(src_hbm.at[q, s], sidx)

        pltpu.make_async_copy(xf_hbm.at[sidx.at[0]], bufs[0], gsems[0]).start()

        def pair(k, carry, first=(ci == 0)):
            j0 = 2 * k
            for b in range(2):
                j = j0 + b
                pltpu.make_async_copy(
                    xf_hbm.at[sidx.at[j]], bufs[b], gsems[b]).wait()

                @pl.when(j + 1 < BPT)
                def _():
                    pltpu.make_async_copy(
                        xf_hbm.at[sidx.at[j + 1]],
                        bufs[1 - b], gsems[1 - b]).start()

                pltpu.sync_copy(bufs[b], accum.at[didx.at[j]], add=True)
                if first:
                    @pl.when(c == 0)
                    def _():
                        pltpu.sync_copy(ones, dacc.at[didx.at[j]], add=True)
            return carry

        lax.fori_loop(0, BPT // 2, pair, 0)

        plsc.subcore_barrier()
        pltpu.sync_copy(accum.at[pl.ds(row0, ROWS)],
                        agg_out.at[q, pl.ds(row0, ROWS)])
        if ci == 0:
            @pl.when(c == 0)
            def _():
                pltpu.sync_copy(dacc.at[pl.ds(row0, ROWS)],
                                deg_out.at[pl.ds(row0, ROWS)])
        if ci < C_PER_CORE - 1:
            for z in range(ROWS // ZROWS):
                pltpu.sync_copy(zbuf, accum.at[pl.ds(row0 + z * ZROWS, ZROWS)])
            plsc.subcore_barrier()


@functools.cache
def _get_sc_segsum():
    mesh = plsc.VectorSubcoreMesh(core_axis_name="c", subcore_axis_name="s")
    return pl.kernel(
        _sc_body,
        mesh=mesh,
        out_type=[
            jax.ShapeDtypeStruct((CHUNKS, N_PAD, FC), jnp.float32),
            jax.ShapeDtypeStruct((N_PAD, DEGW), jnp.float32),
        ],
        scratch_types=[
            pltpu.VMEM((BPT, EBLK), jnp.int32),
            pltpu.VMEM((BPT, EBLK), jnp.int32),
            pltpu.VMEM((EBLK, FC), jnp.float32),
            pltpu.VMEM((EBLK, FC), jnp.float32),
            pltpu.VMEM((ZROWS, FC), jnp.float32),
            pltpu.VMEM((EBLK, DEGW), jnp.float32),
            pltpu.VMEM_SHARED((N_PAD, FC), jnp.float32),
            pltpu.VMEM_SHARED((N_PAD, DEGW), jnp.float32),
            pltpu.SemaphoreType.DMA,
            pltpu.SemaphoreType.DMA,
        ],
        compiler_params=pltpu.CompilerParams(use_tc_tiling_on_sc=False),
    )


def _fold_body(wzl, wzr, wzlin, bzl, bzlin, whl, whr, whlin, bhl, bhlin, attn,
               wcat, bcat, probs):
    mz = wzlin[:F_OUT, :]
    mh = whlin[:F_OUT, :]
    az = jnp.dot(wzl[...], mz, preferred_element_type=jnp.float32)
    bz = jnp.dot(wzr[...], mz, preferred_element_type=jnp.float32)
    ah = jnp.dot(whl[...], mh, preferred_element_type=jnp.float32)
    bh = jnp.dot(whr[...], mh, preferred_element_type=jnp.float32)
    wcat[...] = jnp.concatenate(
        [jnp.concatenate([az, ah], axis=1),
         jnp.concatenate([bz, bh], axis=1)], axis=0)
    cz = jnp.dot(bzl[...], mz, preferred_element_type=jnp.float32) + bzlin[...]
    ch = jnp.dot(bhl[...], mh, preferred_element_type=jnp.float32) + bhlin[...]
    bcat[...] = jnp.concatenate([cz, ch], axis=1)
    a = attn[...]
    e = jnp.exp(a - jnp.max(a))
    probs[...] = e / jnp.sum(e)


_fold = pl.pallas_call(
    _fold_body,
    out_shape=[
        jax.ShapeDtypeStruct((2 * F_IN, 2 * F_OUT), jnp.float32),
        jax.ShapeDtypeStruct((1, 2 * F_OUT), jnp.float32),
        jax.ShapeDtypeStruct((1, PERIODS), jnp.float32),
    ],
)


NB = 10
BN = N // NB


def _main_body(agg, xt, deg, wcat, bcat, probs, w1, b1, w2, b2,
               out, hacc):
    t = pl.program_id(1)
    d = jnp.maximum(deg[:, :1], 1.0)
    mean = jnp.concatenate([agg[0, i] for i in range(NHALF)], axis=1) / d
    xcat = jnp.concatenate([xt[0, i] for i in range(NHALF)], axis=1)
    cat = jnp.concatenate([mean, xcat], axis=1)
    u = jnp.dot(cat, wcat[...], preferred_element_type=jnp.float32) + bcat[...]
    z = jax.nn.sigmoid(u[:, :F_OUT])
    ht = jnp.tanh(u[:, F_OUT:])
    contrib = probs[0, t] * (1.0 - z) * ht

    @pl.when(t == 0)
    def _():
        hacc[...] = contrib

    @pl.when(t > 0)
    def _():
        hacc[...] = hacc[...] + contrib

    @pl.when(t == PERIODS - 1)
    def _():
        h = jnp.maximum(hacc[...], 0.0)
        h1 = jnp.maximum(
            jnp.dot(h, w1[...], preferred_element_type=jnp.float32) + b1[...],
            0.0)
        out[...] = jnp.dot(h1, w2[...], preferred_element_type=jnp.float32) \
            + b2[...]


_main = pl.pallas_call(
    _main_body,
    grid=(NB, PERIODS),
    in_specs=[
        pl.BlockSpec((1, NHALF, BN, FC), lambda nb, t: (t, 0, nb, 0)),
        pl.BlockSpec((1, NHALF, BN, FC), lambda nb, t: (t, 0, nb, 0)),
        pl.BlockSpec((BN, DEGW), lambda nb, t: (nb, 0)),
        pl.BlockSpec((2 * F_IN, 2 * F_OUT), lambda nb, t: (0, 0)),
        pl.BlockSpec((1, 2 * F_OUT), lambda nb, t: (0, 0)),
        pl.BlockSpec(memory_space=pltpu.SMEM),
        pl.BlockSpec((F_OUT, HID), lambda nb, t: (0, 0)),
        pl.BlockSpec((1, HID), lambda nb, t: (0, 0)),
        pl.BlockSpec((HID, OUT_DIM), lambda nb, t: (0, 0)),
        pl.BlockSpec((1, OUT_DIM), lambda nb, t: (0, 0)),
    ],
    out_specs=[
        pl.BlockSpec((BN, OUT_DIM), lambda nb, t: (nb, 0)),
        pl.BlockSpec((BN, F_OUT), lambda nb, t: (nb, 0)),
    ],
    out_shape=[
        jax.ShapeDtypeStruct((N, OUT_DIM), jnp.float32),
        jax.ShapeDtypeStruct((N, F_OUT), jnp.float32),
    ],
)


def kernel(x, edge_index, edge_attr, params):
    p = params
    xq = (x.transpose(2, 1, 0)
          .reshape(PERIODS, NHALF, FC, N)
          .transpose(0, 1, 3, 2))
    xf = xq.reshape(CHUNKS * N, FC)
    src = jnp.concatenate(
        [edge_index[0], jnp.zeros((E_PAD - E,), jnp.int32)])
    dst = jnp.concatenate(
        [edge_index[1], jnp.full((E_PAD - E,), N, jnp.int32)])
    src_off = (src[None, :]
               + (jnp.arange(CHUNKS, dtype=jnp.int32) * N)[:, None])
    src4 = src_off.reshape(CHUNKS, NTILES, BPT, EBLK)
    dst3 = dst.reshape(NTILES, BPT, EBLK)
    zeros_in = jnp.zeros((ZROWS, FC), jnp.float32)
    ones_in = jnp.ones((EBLK, DEGW), jnp.float32)
    zcol = jnp.zeros((ROWS, DEGW), jnp.float32)

    agg, deg = _get_sc_segsum()(src4, dst3, xf, zeros_in, ones_in, zcol)
    agg4 = agg.reshape(PERIODS, NHALF, N_PAD, FC)

    wcat, bcat, probs = _fold(
        p['Wz_l'], p['Wz_r'], p['Wz_lin'],
        p['bz_l'].reshape(1, F_OUT), p['bz_lin'].reshape(1, F_OUT),
        p['Wh_l'], p['Wh_r'], p['Wh_lin'],
        p['bh_l'].reshape(1, F_OUT), p['bh_lin'].reshape(1, F_OUT),
        p['attn'].reshape(1, PERIODS))

    out, hacc = _main(agg4, xq, deg, wcat, bcat, probs,
                      p['W1'], p['b1'].reshape(1, HID),
                      p['W2'], p['b2'].reshape(1, OUT_DIM))
    return (out, hacc)


# speedup vs baseline: 1.1149x; 1.1149x over previous
import functools

import jax
import jax.numpy as jnp
from jax import lax
from jax.experimental import pallas as pl
from jax.experimental.pallas import tpu as pltpu
from jax.experimental.pallas import tpu_sc as plsc

N = 10000
E = 320000
F_IN = 128
F_OUT = 256
PERIODS = 12
HID = 128
OUT_DIM = 1

NCORES = 2
NTILES = 16
EBLK = 96
BPT = 210
E_PAD = NTILES * BPT * EBLK
FC = 64
NHALF = F_IN // FC
CHUNKS = PERIODS * NHALF
C_PER_CORE = CHUNKS // NCORES
N_PAD = 10240
ROWS = N_PAD // NTILES
ZROWS = 128
DEGW = 16


def _sc_body(src_hbm, dst_hbm, xf_hbm, zeros_hbm, ones_hbm, zcol_hbm,
             agg_out, deg_out,
             sidx, didx, rows_a, rows_b, zbuf, ones, accum, dacc,
             gsem_a, gsem_b):
    c = lax.axis_index("c")
    s = lax.axis_index("s")
    row0 = s * ROWS
    bufs = (rows_a, rows_b)
    gsems = (gsem_a, gsem_b)

    pltpu.sync_copy(zeros_hbm, zbuf)
    pltpu.sync_copy(ones_hbm, ones)
    pltpu.sync_copy(dst_hbm.at[s], didx)
    for z in range(ROWS // ZROWS):
        pltpu.sync_copy(zbuf, accum.at[pl.ds(row0 + z * ZROWS, ZROWS)])

    @pl.when(c == 0)
    def _():
        pltpu.sync_copy(zcol_hbm, dacc.at[pl.ds(row0, ROWS)])

    plsc.subcore_barrier()

    for ci in range(C_PER_CORE):
        q = c * C_PER_CORE + ci
        pltpu.sync_copy(src_hbm.at[q, s], sidx)

        pltpu.make_async_copy(xf_hbm.at[sidx.at[0]], bufs[0], gsems[0]).start()

        def pair(k, carry, first=(ci == 0)):
            j0 = 2 * k
            for b in range(2):
                j = j0 + b
                pltpu.make_async_copy(
                    xf_hbm.at[sidx.at[j]], bufs[b], gsems[b]).wait()

                @pl.when(j + 1 < BPT)
                def _():
                    pltpu.make_async_copy(
                        xf_hbm.at[sidx.at[j + 1]],
                        bufs[1 - b], gsems[1 - b]).start()

                pltpu.sync_copy(bufs[b], accum.at[didx.at[j]], add=True)
                if first:
                    @pl.when(c == 0)
                    def _():
                        pltpu.sync_copy(ones, dacc.at[didx.at[j]], add=True)
            return carry

        lax.fori_loop(0, BPT // 2, pair, 0)

        plsc.subcore_barrier()
        pltpu.sync_copy(accum.at[pl.ds(row0, ROWS)],
                        agg_out.at[q, pl.ds(row0, ROWS)])
        if ci == 0:
            @pl.when(c == 0)
            def _():
                pltpu.sync_copy(dacc.at[pl.ds(row0, ROWS)],
                                deg_out.at[pl.ds(row0, ROWS)])
        if ci < C_PER_CORE - 1:
            for z in range(ROWS // ZROWS):
                pltpu.sync_copy(zbuf, accum.at[pl.ds(row0 + z * ZROWS, ZROWS)])
            plsc.subcore_barrier()


@functools.cache
def _get_sc_segsum():
    mesh = plsc.VectorSubcoreMesh(core_axis_name="c", subcore_axis_name="s")
    return pl.kernel(
        _sc_body,
        mesh=mesh,
        out_type=[
            jax.ShapeDtypeStruct((CHUNKS, N_PAD, FC), jnp.float32),
            jax.ShapeDtypeStruct((N_PAD, DEGW), jnp.float32),
        ],
        scratch_types=[
            pltpu.VMEM((BPT, EBLK), jnp.int32),
            pltpu.VMEM((BPT, EBLK), jnp.int32),
            pltpu.VMEM((EBLK, FC), jnp.float32),
            pltpu.VMEM((EBLK, FC), jnp.float32),
            pltpu.VMEM((ZROWS, FC), jnp.float32),
            pltpu.VMEM((EBLK, DEGW), jnp.float32),
            pltpu.VMEM_SHARED((N_PAD, FC), jnp.float32),
            pltpu.VMEM_SHARED((N_PAD, DEGW), jnp.float32),
            pltpu.SemaphoreType.DMA,
            pltpu.SemaphoreType.DMA,
        ],
        compiler_params=pltpu.CompilerParams(use_tc_tiling_on_sc=False),
    )


def _fold_body(wzl, wzr, wzlin, bzl, bzlin, whl, whr, whlin, bhl, bhlin, attn,
               wcat, bcat, probs):
    mz = wzlin[:F_OUT, :]
    mh = whlin[:F_OUT, :]
    az = jnp.dot(wzl[...], mz, preferred_element_type=jnp.float32)
    bz = jnp.dot(wzr[...], mz, preferred_element_type=jnp.float32)
    ah = jnp.dot(whl[...], mh, preferred_element_type=jnp.float32)
    bh = jnp.dot(whr[...], mh, preferred_element_type=jnp.float32)
    wcat[...] = jnp.concatenate(
        [jnp.concatenate([az, ah], axis=1),
         jnp.concatenate([bz, bh], axis=1)], axis=0)
    cz = jnp.dot(bzl[...], mz, preferred_element_type=jnp.float32) + bzlin[...]
    ch = jnp.dot(bhl[...], mh, preferred_element_type=jnp.float32) + bhlin[...]
    bcat[...] = jnp.concatenate([cz, ch], axis=1)
    a = attn[...]
    e = jnp.exp(a - jnp.max(a))
    probs[...] = e / jnp.sum(e)


_fold = pl.pallas_call(
    _fold_body,
    out_shape=[
        jax.ShapeDtypeStruct((2 * F_IN, 2 * F_OUT), jnp.float32),
        jax.ShapeDtypeStruct((1, 2 * F_OUT), jnp.float32),
        jax.ShapeDtypeStruct((1, PERIODS), jnp.float32),
    ],
)


NB = 10
BN = N // NB


def _main_body(agg, xt, deg, wcat, bcat, probs, w1, b1, w2, b2,
               out, hacc):
    t = pl.program_id(1)
    d = jnp.maximum(deg[:, :1], 1.0)
    mean = jnp.concatenate([agg[0, i] for i in range(NHALF)], axis=1) / d
    xcat = jnp.concatenate([xt[0, i] for i in range(NHALF)], axis=1)
    cat = jnp.concatenate([mean, xcat], axis=1)
    u = jnp.dot(cat, wcat[...], preferred_element_type=jnp.float32) + bcat[...]
    z = jax.nn.sigmoid(u[:, :F_OUT])
    ht = jnp.tanh(u[:, F_OUT:])
    contrib = probs[0, t] * (1.0 - z) * ht

    @pl.when(t == 0)
    def _():
        hacc[...] = contrib

    @pl.when(t > 0)
    def _():
        hacc[...] = hacc[...] + contrib

    @pl.when(t == PERIODS - 1)
    def _():
        h = jnp.maximum(hacc[...], 0.0)
        h1 = jnp.maximum(
            jnp.dot(h, w1[...], preferred_element_type=jnp.float32) + b1[...],
            0.0)
        out[...] = jnp.dot(h1, w2[...], preferred_element_type=jnp.float32) \
            + b2[...]


_main = pl.pallas_call(
    _main_body,
    grid=(NB, PERIODS),
    in_specs=[
        pl.BlockSpec((1, NHALF, BN, FC), lambda nb, t: (t, 0, nb, 0)),
        pl.BlockSpec((1, NHALF, BN, FC), lambda nb, t: (t, 0, nb, 0)),
        pl.BlockSpec((BN, DEGW), lambda nb, t: (nb, 0)),
        pl.BlockSpec((2 * F_IN, 2 * F_OUT), lambda nb, t: (0, 0)),
        pl.BlockSpec((1, 2 * F_OUT), lambda nb, t: (0, 0)),
        pl.BlockSpec(memory_space=pltpu.SMEM),
        pl.BlockSpec((F_OUT, HID), lambda nb, t: (0, 0)),
        pl.BlockSpec((1, HID), lambda nb, t: (0, 0)),
        pl.BlockSpec((HID, OUT_DIM), lambda nb, t: (0, 0)),
        pl.BlockSpec((1, OUT_DIM), lambda nb, t: (0, 0)),
    ],
    out_specs=[
        pl.BlockSpec((BN, OUT_DIM), lambda nb, t: (nb, 0)),
        pl.BlockSpec((BN, F_OUT), lambda nb, t: (nb, 0)),
    ],
    out_shape=[
        jax.ShapeDtypeStruct((N, OUT_DIM), jnp.float32),
        jax.ShapeDtypeStruct((N, F_OUT), jnp.float32),
    ],
)


def kernel(x, edge_index, edge_attr, params):
    p = params
    xq = (x.transpose(2, 1, 0)
          .reshape(PERIODS, NHALF, FC, N)
          .transpose(0, 1, 3, 2))
    xf = xq.reshape(CHUNKS * N, FC)
    src = jnp.concatenate(
        [edge_index[0], jnp.zeros((E_PAD - E,), jnp.int32)])
    dst = jnp.concatenate(
        [edge_index[1], jnp.full((E_PAD - E,), N, jnp.int32)])
    src_off = (src[None, :]
               + (jnp.arange(CHUNKS, dtype=jnp.int32) * N)[:, None])
    src4 = src_off.reshape(CHUNKS, NTILES, BPT, EBLK)
    dst3 = dst.reshape(NTILES, BPT, EBLK)
    zeros_in = jnp.zeros((ZROWS, FC), jnp.float32)
    ones_in = jnp.ones((EBLK, DEGW), jnp.float32)
    zcol = jnp.zeros((ROWS, DEGW), jnp.float32)

    agg, deg = _get_sc_segsum()(src4, dst3, xf, zeros_in, ones_in, zcol)
    agg4 = agg.reshape(PERIODS, NHALF, N_PAD, FC)

    wcat, bcat, probs = _fold(
        p['Wz_l'], p['Wz_r'], p['Wz_lin'],
        p['bz_l'].reshape(1, F_OUT), p['bz_lin'].reshape(1, F_OUT),
        p['Wh_l'], p['Wh_r'], p['Wh_lin'],
        p['bh_l'].reshape(1, F_OUT), p['bh_lin'].reshape(1, F_OUT),
        p['attn'].reshape(1, PERIODS))

    out, hacc = _main(agg4, xq, deg, wcat, bcat, probs,
                      p['W1'], p['b1'].reshape(1, HID),
                      p['W2'], p['b2'].reshape(1, OUT_DIM))
    return (out, hacc)


# speedup vs baseline: 1.1310x; 1.0145x over previous
import functools

import jax
import jax.numpy as jnp
from jax import lax
from jax.experimental import pallas as pl
from jax.experimental.pallas import tpu as pltpu
from jax.experimental.pallas import tpu_sc as plsc

N = 10000
E = 320000
F_IN = 128
F_OUT = 256
PERIODS = 12
HID = 128
OUT_DIM = 1

NCORES = 2
NTILES = 16
EBLK = 128
BPT = 158
E_PAD = NTILES * BPT * EBLK
FC = 64
NHALF = F_IN // FC
CHUNKS = PERIODS * NHALF
C_PER_CORE = CHUNKS // NCORES
N_PAD = 10240
ROWS = N_PAD // NTILES
ZROWS = 128
DEGW = 16


def _sc_body(src_hbm, dst_hbm, xf_hbm, zeros_hbm, ones_hbm, zcol_hbm,
             agg_out, deg_out,
             sidx, didx, rows_a, rows_b, zbuf, ones, accum, dacc,
             gsem_a, gsem_b):
    c = lax.axis_index("c")
    s = lax.axis_index("s")
    row0 = s * ROWS
    bufs = (rows_a, rows_b)
    gsems = (gsem_a, gsem_b)

    pltpu.sync_copy(zeros_hbm, zbuf)
    pltpu.sync_copy(ones_hbm, ones)
    pltpu.sync_copy(dst_hbm.at[s], didx)
    for z in range(ROWS // ZROWS):
        pltpu.sync_copy(zbuf, accum.at[pl.ds(row0 + z * ZROWS, ZROWS)])

    @pl.when(c == 0)
    def _():
        pltpu.sync_copy(zcol_hbm, dacc.at[pl.ds(row0, ROWS)])

    plsc.subcore_barrier()

    for ci in range(C_PER_CORE):
        q = c * C_PER_CORE + ci
        pltpu.sync_copy(src_hbm.at[q, s], sidx)

        pltpu.make_async_copy(xf_hbm.at[sidx.at[0]], bufs[0], gsems[0]).start()

        def pair(k, carry, first=(ci == 0)):
            j0 = 2 * k
            for b in range(2):
                j = j0 + b
                pltpu.make_async_copy(
                    xf_hbm.at[sidx.at[j]], bufs[b], gsems[b]).wait()

                @pl.when(j + 1 < BPT)
                def _():
                    pltpu.make_async_copy(
                        xf_hbm.at[sidx.at[j + 1]],
                        bufs[1 - b], gsems[1 - b]).start()

                pltpu.sync_copy(bufs[b], accum.at[didx.at[j]], add=True)
                if first:
                    @pl.when(c == 0)
                    def _():
                        pltpu.sync_copy(ones, dacc.at[didx.at[j]], add=True)
            return carry

        lax.fori_loop(0, BPT // 2, pair, 0)

        plsc.subcore_barrier()
        pltpu.sync_copy(accum.at[pl.ds(row0, ROWS)],
                        agg_out.at[q, pl.ds(row0, ROWS)])
        if ci == 0:
            @pl.when(c == 0)
            def _():
                pltpu.sync_copy(dacc.at[pl.ds(row0, ROWS)],
                                deg_out.at[pl.ds(row0, ROWS)])
        if ci < C_PER_CORE - 1:
            for z in range(ROWS // ZROWS):
                pltpu.sync_copy(zbuf, accum.at[pl.ds(row0 + z * ZROWS, ZROWS)])
            plsc.subcore_barrier()


@functools.cache
def _get_sc_segsum():
    mesh = plsc.VectorSubcoreMesh(core_axis_name="c", subcore_axis_name="s")
    return pl.kernel(
        _sc_body,
        mesh=mesh,
        out_type=[
            jax.ShapeDtypeStruct((CHUNKS, N_PAD, FC), jnp.float32),
            jax.ShapeDtypeStruct((N_PAD, DEGW), jnp.float32),
        ],
        scratch_types=[
            pltpu.VMEM((BPT, EBLK), jnp.int32),
            pltpu.VMEM((BPT, EBLK), jnp.int32),
            pltpu.VMEM((EBLK, FC), jnp.float32),
            pltpu.VMEM((EBLK, FC), jnp.float32),
            pltpu.VMEM((ZROWS, FC), jnp.float32),
            pltpu.VMEM((EBLK, DEGW), jnp.float32),
            pltpu.VMEM_SHARED((N_PAD, FC), jnp.float32),
            pltpu.VMEM_SHARED((N_PAD, DEGW), jnp.float32),
            pltpu.SemaphoreType.DMA,
            pltpu.SemaphoreType.DMA,
        ],
        compiler_params=pltpu.CompilerParams(use_tc_tiling_on_sc=False),
    )


def _fold_body(wzl, wzr, wzlin, bzl, bzlin, whl, whr, whlin, bhl, bhlin, attn,
               wcat, bcat, probs):
    mz = wzlin[:F_OUT, :]
    mh = whlin[:F_OUT, :]
    az = jnp.dot(wzl[...], mz, preferred_element_type=jnp.float32)
    bz = jnp.dot(wzr[...], mz, preferred_element_type=jnp.float32)
    ah = jnp.dot(whl[...], mh, preferred_element_type=jnp.float32)
    bh = jnp.dot(whr[...], mh, preferred_element_type=jnp.float32)
    wcat[...] = jnp.concatenate(
        [jnp.concatenate([az, ah], axis=1),
         jnp.concatenate([bz, bh], axis=1)], axis=0)
    cz = jnp.dot(bzl[...], mz, preferred_element_type=jnp.float32) + bzlin[...]
    ch = jnp.dot(bhl[...], mh, preferred_element_type=jnp.float32) + bhlin[...]
    bcat[...] = jnp.concatenate([cz, ch], axis=1)
    a = attn[...]
    e = jnp.exp(a - jnp.max(a))
    probs[...] = e / jnp.sum(e)


_fold = pl.pallas_call(
    _fold_body,
    out_shape=[
        jax.ShapeDtypeStruct((2 * F_IN, 2 * F_OUT), jnp.float32),
        jax.ShapeDtypeStruct((1, 2 * F_OUT), jnp.float32),
        jax.ShapeDtypeStruct((1, PERIODS), jnp.float32),
    ],
)


NB = 10
BN = N // NB


def _main_body(agg, xt, deg, wcat, bcat, probs, w1, b1, w2, b2,
               out, hacc):
    t = pl.program_id(1)
    d = jnp.maximum(deg[:, :1], 1.0)
    mean = jnp.concatenate([agg[0, i] for i in range(NHALF)], axis=1) / d
    xcat = jnp.concatenate([xt[0, i] for i in range(NHALF)], axis=1)
    cat = jnp.concatenate([mean, xcat], axis=1)
    u = jnp.dot(cat, wcat[...], preferred_element_type=jnp.float32) + bcat[...]
    z = jax.nn.sigmoid(u[:, :F_OUT])
    ht = jnp.tanh(u[:, F_OUT:])
    contrib = probs[0, t] * (1.0 - z) * ht

    @pl.when(t == 0)
    def _():
        hacc[...] = contrib

    @pl.when(t > 0)
    def _():
        hacc[...] = hacc[...] + contrib

    @pl.when(t == PERIODS - 1)
    def _():
        h = jnp.maximum(hacc[...], 0.0)
        h1 = jnp.maximum(
            jnp.dot(h, w1[...], preferred_element_type=jnp.float32) + b1[...],
            0.0)
        out[...] = jnp.dot(h1, w2[...], preferred_element_type=jnp.float32) \
            + b2[...]


_main = pl.pallas_call(
    _main_body,
    grid=(NB, PERIODS),
    in_specs=[
        pl.BlockSpec((1, NHALF, BN, FC), lambda nb, t: (t, 0, nb, 0)),
        pl.BlockSpec((1, NHALF, BN, FC), lambda nb, t: (t, 0, nb, 0)),
        pl.BlockSpec((BN, DEGW), lambda nb, t: (nb, 0)),
        pl.BlockSpec((2 * F_IN, 2 * F_OUT), lambda nb, t: (0, 0)),
        pl.BlockSpec((1, 2 * F_OUT), lambda nb, t: (0, 0)),
        pl.BlockSpec(memory_space=pltpu.SMEM),
        pl.BlockSpec((F_OUT, HID), lambda nb, t: (0, 0)),
        pl.BlockSpec((1, HID), lambda nb, t: (0, 0)),
        pl.BlockSpec((HID, OUT_DIM), lambda nb, t: (0, 0)),
        pl.BlockSpec((1, OUT_DIM), lambda nb, t: (0, 0)),
    ],
    out_specs=[
        pl.BlockSpec((BN, OUT_DIM), lambda nb, t: (nb, 0)),
        pl.BlockSpec((BN, F_OUT), lambda nb, t: (nb, 0)),
    ],
    out_shape=[
        jax.ShapeDtypeStruct((N, OUT_DIM), jnp.float32),
        jax.ShapeDtypeStruct((N, F_OUT), jnp.float32),
    ],
)


def kernel(x, edge_index, edge_attr, params):
    p = params
    xq = (x.transpose(2, 1, 0)
          .reshape(PERIODS, NHALF, FC, N)
          .transpose(0, 1, 3, 2))
    xf = xq.reshape(CHUNKS * N, FC)
    src = jnp.concatenate(
        [edge_index[0], jnp.zeros((E_PAD - E,), jnp.int32)])
    dst = jnp.concatenate(
        [edge_index[1],
         N + jnp.arange(E_PAD - E, dtype=jnp.int32) % (N_PAD - N)])
    src_off = (src[None, :]
               + (jnp.arange(CHUNKS, dtype=jnp.int32) * N)[:, None])
    src4 = src_off.reshape(CHUNKS, NTILES, BPT, EBLK)
    dst3 = dst.reshape(NTILES, BPT, EBLK)
    zeros_in = jnp.zeros((ZROWS, FC), jnp.float32)
    ones_in = jnp.ones((EBLK, DEGW), jnp.float32)
    zcol = jnp.zeros((ROWS, DEGW), jnp.float32)

    agg, deg = _get_sc_segsum()(src4, dst3, xf, zeros_in, ones_in, zcol)
    agg4 = agg.reshape(PERIODS, NHALF, N_PAD, FC)

    wcat, bcat, probs = _fold(
        p['Wz_l'], p['Wz_r'], p['Wz_lin'],
        p['bz_l'].reshape(1, F_OUT), p['bz_lin'].reshape(1, F_OUT),
        p['Wh_l'], p['Wh_r'], p['Wh_lin'],
        p['bh_l'].reshape(1, F_OUT), p['bh_lin'].reshape(1, F_OUT),
        p['attn'].reshape(1, PERIODS))

    out, hacc = _main(agg4, xq, deg, wcat, bcat, probs,
                      p['W1'], p['b1'].reshape(1, HID),
                      p['W2'], p['b2'].reshape(1, OUT_DIM))
    return (out, hacc)


# speedup vs baseline: 1.2383x; 1.0949x over previous
import functools

import jax
import jax.numpy as jnp
from jax import lax
from jax.experimental import pallas as pl
from jax.experimental.pallas import tpu as pltpu
from jax.experimental.pallas import tpu_sc as plsc

N = 10000
E = 320000
F_IN = 128
F_OUT = 256
PERIODS = 12
HID = 128
OUT_DIM = 1

NCORES = 2
NTILES = 16
EBLK = 80
BPT = 250
E_PAD = NTILES * BPT * EBLK
FC = 64
NHALF = F_IN // FC
CHUNKS = PERIODS * NHALF
C_PER_CORE = CHUNKS // NCORES
N_PAD = 10240
ROWS = N_PAD // NTILES
ZROWS = 128
DEGW = 16


def _sc_body(src_hbm, dst_hbm, xf_hbm, zeros_hbm, ones_hbm, zcol_hbm,
             agg_out, deg_out,
             sidx, didx, rows_a, rows_b, zbuf, ones, accum, dacc,
             gsem_a, gsem_b):
    c = lax.axis_index("c")
    s = lax.axis_index("s")
    row0 = s * ROWS
    bufs = (rows_a, rows_b)
    gsems = (gsem_a, gsem_b)

    pltpu.sync_copy(zeros_hbm, zbuf)
    pltpu.sync_copy(ones_hbm, ones)
    pltpu.sync_copy(dst_hbm.at[s], didx)
    for z in range(ROWS // ZROWS):
        pltpu.sync_copy(zbuf, accum.at[pl.ds(row0 + z * ZROWS, ZROWS)])

    @pl.when(c == 0)
    def _():
        pltpu.sync_copy(zcol_hbm, dacc.at[pl.ds(row0, ROWS)])

    plsc.subcore_barrier()

    for ci in range(C_PER_CORE):
        q = c * C_PER_CORE + ci
        pltpu.sync_copy(src_hbm.at[q, s], sidx)

        pltpu.make_async_copy(xf_hbm.at[sidx.at[0]], bufs[0], gsems[0]).start()

        def pair(k, carry, first=(ci == 0)):
            j0 = 2 * k
            for b in range(2):
                j = j0 + b
                pltpu.make_async_copy(
                    xf_hbm.at[sidx.at[j]], bufs[b], gsems[b]).wait()

                @pl.when(j + 1 < BPT)
                def _():
                    pltpu.make_async_copy(
                        xf_hbm.at[sidx.at[j + 1]],
                        bufs[1 - b], gsems[1 - b]).start()

                pltpu.sync_copy(bufs[b], accum.at[didx.at[j]], add=True)
                if first:
                    @pl.when(c == 0)
                    def _():
                        pltpu.sync_copy(ones, dacc.at[didx.at[j]], add=True)
            return carry

        lax.fori_loop(0, BPT // 2, pair, 0)

        plsc.subcore_barrier()
        pltpu.sync_copy(accum.at[pl.ds(row0, ROWS)],
                        agg_out.at[q, pl.ds(row0, ROWS)])
        if ci == 0:
            @pl.when(c == 0)
            def _():
                pltpu.sync_copy(dacc.at[pl.ds(row0, ROWS)],
                                deg_out.at[pl.ds(row0, ROWS)])
        if ci < C_PER_CORE - 1:
            for z in range(ROWS // ZROWS):
                pltpu.sync_copy(zbuf, accum.at[pl.ds(row0 + z * ZROWS, ZROWS)])
            plsc.subcore_barrier()


@functools.cache
def _get_sc_segsum():
    mesh = plsc.VectorSubcoreMesh(core_axis_name="c", subcore_axis_name="s")
    return pl.kernel(
        _sc_body,
        mesh=mesh,
        out_type=[
            jax.ShapeDtypeStruct((CHUNKS, N_PAD, FC), jnp.float32),
            jax.ShapeDtypeStruct((N_PAD, DEGW), jnp.float32),
        ],
        scratch_types=[
            pltpu.VMEM((BPT, EBLK), jnp.int32),
            pltpu.VMEM((BPT, EBLK), jnp.int32),
            pltpu.VMEM((EBLK, FC), jnp.float32),
            pltpu.VMEM((EBLK, FC), jnp.float32),
            pltpu.VMEM((ZROWS, FC), jnp.float32),
            pltpu.VMEM((EBLK, DEGW), jnp.float32),
            pltpu.VMEM_SHARED((N_PAD, FC), jnp.float32),
            pltpu.VMEM_SHARED((N_PAD, DEGW), jnp.float32),
            pltpu.SemaphoreType.DMA,
            pltpu.SemaphoreType.DMA,
        ],
        compiler_params=pltpu.CompilerParams(use_tc_tiling_on_sc=False),
    )


def _fold_body(wzl, wzr, wzlin, bzl, bzlin, whl, whr, whlin, bhl, bhlin, attn,
               wcat, bcat, probs):
    mz = wzlin[:F_OUT, :]
    mh = whlin[:F_OUT, :]
    az = jnp.dot(wzl[...], mz, preferred_element_type=jnp.float32)
    bz = jnp.dot(wzr[...], mz, preferred_element_type=jnp.float32)
    ah = jnp.dot(whl[...], mh, preferred_element_type=jnp.float32)
    bh = jnp.dot(whr[...], mh, preferred_element_type=jnp.float32)
    wcat[...] = jnp.concatenate(
        [jnp.concatenate([az, ah], axis=1),
         jnp.concatenate([bz, bh], axis=1)], axis=0)
    cz = jnp.dot(bzl[...], mz, preferred_element_type=jnp.float32) + bzlin[...]
    ch = jnp.dot(bhl[...], mh, preferred_element_type=jnp.float32) + bhlin[...]
    bcat[...] = jnp.concatenate([cz, ch], axis=1)
    a = attn[...]
    e = jnp.exp(a - jnp.max(a))
    probs[...] = e / jnp.sum(e)


_fold = pl.pallas_call(
    _fold_body,
    out_shape=[
        jax.ShapeDtypeStruct((2 * F_IN, 2 * F_OUT), jnp.float32),
        jax.ShapeDtypeStruct((1, 2 * F_OUT), jnp.float32),
        jax.ShapeDtypeStruct((1, PERIODS), jnp.float32),
    ],
)


NB = 10
BN = N // NB


def _main_body(agg, xt, deg, wcat, bcat, probs, w1, b1, w2, b2,
               out, hacc):
    t = pl.program_id(1)
    d = jnp.maximum(deg[:, :1], 1.0)
    mean = jnp.concatenate([agg[0, i] for i in range(NHALF)], axis=1) / d
    xcat = jnp.concatenate([xt[0, i] for i in range(NHALF)], axis=1)
    cat = jnp.concatenate([mean, xcat], axis=1)
    u = jnp.dot(cat, wcat[...], preferred_element_type=jnp.float32) + bcat[...]
    z = jax.nn.sigmoid(u[:, :F_OUT])
    ht = jnp.tanh(u[:, F_OUT:])
    contrib = probs[0, t] * (1.0 - z) * ht

    @pl.when(t == 0)
    def _():
        hacc[...] = contrib

    @pl.when(t > 0)
    def _():
        hacc[...] = hacc[...] + contrib

    @pl.when(t == PERIODS - 1)
    def _():
        h = jnp.maximum(hacc[...], 0.0)
        h1 = jnp.maximum(
            jnp.dot(h, w1[...], preferred_element_type=jnp.float32) + b1[...],
            0.0)
        out[...] = jnp.dot(h1, w2[...], preferred_element_type=jnp.float32) \
            + b2[...]


_main = pl.pallas_call(
    _main_body,
    grid=(NB, PERIODS),
    in_specs=[
        pl.BlockSpec((1, NHALF, BN, FC), lambda nb, t: (t, 0, nb, 0)),
        pl.BlockSpec((1, NHALF, BN, FC), lambda nb, t: (t, 0, nb, 0)),
        pl.BlockSpec((BN, DEGW), lambda nb, t: (nb, 0)),
        pl.BlockSpec((2 * F_IN, 2 * F_OUT), lambda nb, t: (0, 0)),
        pl.BlockSpec((1, 2 * F_OUT), lambda nb, t: (0, 0)),
        pl.BlockSpec(memory_space=pltpu.SMEM),
        pl.BlockSpec((F_OUT, HID), lambda nb, t: (0, 0)),
        pl.BlockSpec((1, HID), lambda nb, t: (0, 0)),
        pl.BlockSpec((HID, OUT_DIM), lambda nb, t: (0, 0)),
        pl.BlockSpec((1, OUT_DIM), lambda nb, t: (0, 0)),
    ],
    out_specs=[
        pl.BlockSpec((BN, OUT_DIM), lambda nb, t: (nb, 0)),
        pl.BlockSpec((BN, F_OUT), lambda nb, t: (nb, 0)),
    ],
    out_shape=[
        jax.ShapeDtypeStruct((N, OUT_DIM), jnp.float32),
        jax.ShapeDtypeStruct((N, F_OUT), jnp.float32),
    ],
)


def kernel(x, edge_index, edge_attr, params):
    p = params
    xq = (x.transpose(2, 1, 0)
          .reshape(PERIODS, NHALF, FC, N)
          .transpose(0, 1, 3, 2))
    xf = xq.reshape(CHUNKS * N, FC)
    src = jnp.concatenate(
        [edge_index[0], jnp.zeros((E_PAD - E,), jnp.int32)])
    dst = jnp.concatenate(
        [edge_index[1],
         N + jnp.arange(E_PAD - E, dtype=jnp.int32) % (N_PAD - N)])
    src_off = (src[None, :]
               + (jnp.arange(CHUNKS, dtype=jnp.int32) * N)[:, None])
    src4 = src_off.reshape(CHUNKS, NTILES, BPT, EBLK)
    dst3 = dst.reshape(NTILES, BPT, EBLK)
    zeros_in = jnp.zeros((ZROWS, FC), jnp.float32)
    ones_in = jnp.ones((EBLK, DEGW), jnp.float32)
    zcol = jnp.zeros((ROWS, DEGW), jnp.float32)

    agg, deg = _get_sc_segsum()(src4, dst3, xf, zeros_in, ones_in, zcol)
    agg4 = agg.reshape(PERIODS, NHALF, N_PAD, FC)

    wcat, bcat, probs = _fold(
        p['Wz_l'], p['Wz_r'], p['Wz_lin'],
        p['bz_l'].reshape(1, F_OUT), p['bz_lin'].reshape(1, F_OUT),
        p['Wh_l'], p['Wh_r'], p['Wh_lin'],
        p['bh_l'].reshape(1, F_OUT), p['bh_lin'].reshape(1, F_OUT),
        p['attn'].reshape(1, PERIODS))

    out, hacc = _main(agg4, xq, deg, wcat, bcat, probs,
                      p['W1'], p['b1'].reshape(1, HID),
                      p['W2'], p['b2'].reshape(1, OUT_DIM))
    return (out, hacc)


# speedup vs baseline: 1.2646x; 1.0213x over previous
import functools

import jax
import jax.numpy as jnp
from jax import lax
from jax.experimental import pallas as pl
from jax.experimental.pallas import tpu as pltpu
from jax.experimental.pallas import tpu_sc as plsc

N = 10000
E = 320000
F_IN = 128
F_OUT = 256
PERIODS = 12
HID = 128
OUT_DIM = 1

NCORES = 2
NTILES = 16
EBLK = 80
BPT = 250
E_PAD = NTILES * BPT * EBLK
FC = 64
NHALF = F_IN // FC
CHUNKS = PERIODS * NHALF
C_PER_CORE = CHUNKS // NCORES
N_PAD = 10240
ROWS = N_PAD // NTILES
ZROWS = 128
DEGW = 16


def _sc_body(src_hbm, dst_hbm, xf_hbm, zeros_hbm, ones_hbm, zcol_hbm,
             agg_out, deg_out,
             sidx, didx, rows_a, rows_b, zbuf, ones, accum, dacc,
             gsem_a, gsem_b):
    c = lax.axis_index("c")
    s = lax.axis_index("s")
    row0 = s * ROWS
    bufs = (rows_a, rows_b)
    gsems = (gsem_a, gsem_b)

    pltpu.sync_copy(zeros_hbm, zbuf)
    pltpu.sync_copy(ones_hbm, ones)
    pltpu.sync_copy(src_hbm.at[s], sidx)
    pltpu.sync_copy(dst_hbm.at[s], didx)
    for z in range(ROWS // ZROWS):
        pltpu.sync_copy(zbuf, accum.at[pl.ds(row0 + z * ZROWS, ZROWS)])

    @pl.when(c == 0)
    def _():
        pltpu.sync_copy(zcol_hbm, dacc.at[pl.ds(row0, ROWS)])

    plsc.subcore_barrier()

    for ci in range(C_PER_CORE):
        q = c * C_PER_CORE + ci
        xq_hbm = xf_hbm.at[pl.ds(q * N, N)]

        pltpu.make_async_copy(xq_hbm.at[sidx.at[0]], bufs[0], gsems[0]).start()

        def pair(k, carry, first=(ci == 0)):
            j0 = 2 * k
            for b in range(2):
                j = j0 + b
                pltpu.make_async_copy(
                    xq_hbm.at[sidx.at[j]], bufs[b], gsems[b]).wait()

                @pl.when(j + 1 < BPT)
                def _():
                    pltpu.make_async_copy(
                        xq_hbm.at[sidx.at[j + 1]],
                        bufs[1 - b], gsems[1 - b]).start()

                pltpu.sync_copy(bufs[b], accum.at[didx.at[j]], add=True)
                if first:
                    @pl.when(c == 0)
                    def _():
                        pltpu.sync_copy(ones, dacc.at[didx.at[j]], add=True)
            return carry

        lax.fori_loop(0, BPT // 2, pair, 0)

        plsc.subcore_barrier()
        pltpu.sync_copy(accum.at[pl.ds(row0, ROWS)],
                        agg_out.at[q, pl.ds(row0, ROWS)])
        if ci == 0:
            @pl.when(c == 0)
            def _():
                pltpu.sync_copy(dacc.at[pl.ds(row0, ROWS)],
                                deg_out.at[pl.ds(row0, ROWS)])
        if ci < C_PER_CORE - 1:
            for z in range(ROWS // ZROWS):
                pltpu.sync_copy(zbuf, accum.at[pl.ds(row0 + z * ZROWS, ZROWS)])
            plsc.subcore_barrier()


@functools.cache
def _get_sc_segsum():
    mesh = plsc.VectorSubcoreMesh(core_axis_name="c", subcore_axis_name="s")
    return pl.kernel(
        _sc_body,
        mesh=mesh,
        out_type=[
            jax.ShapeDtypeStruct((CHUNKS, N_PAD, FC), jnp.float32),
            jax.ShapeDtypeStruct((N_PAD, DEGW), jnp.float32),
        ],
        scratch_types=[
            pltpu.VMEM((BPT, EBLK), jnp.int32),
            pltpu.VMEM((BPT, EBLK), jnp.int32),
            pltpu.VMEM((EBLK, FC), jnp.float32),
            pltpu.VMEM((EBLK, FC), jnp.float32),
            pltpu.VMEM((ZROWS, FC), jnp.float32),
            pltpu.VMEM((EBLK, DEGW), jnp.float32),
            pltpu.VMEM_SHARED((N_PAD, FC), jnp.float32),
            pltpu.VMEM_SHARED((N_PAD, DEGW), jnp.float32),
            pltpu.SemaphoreType.DMA,
            pltpu.SemaphoreType.DMA,
        ],
        compiler_params=pltpu.CompilerParams(use_tc_tiling_on_sc=False),
    )


def _fold_body(wzl, wzr, wzlin, bzl, bzlin, whl, whr, whlin, bhl, bhlin, attn,
               wcat, bcat, probs):
    mz = wzlin[:F_OUT, :]
    mh = whlin[:F_OUT, :]
    az = jnp.dot(wzl[...], mz, preferred_element_type=jnp.float32)
    bz = jnp.dot(wzr[...], mz, preferred_element_type=jnp.float32)
    ah = jnp.dot(whl[...], mh, preferred_element_type=jnp.float32)
    bh = jnp.dot(whr[...], mh, preferred_element_type=jnp.float32)
    wcat[...] = jnp.concatenate(
        [jnp.concatenate([az, ah], axis=1),
         jnp.concatenate([bz, bh], axis=1)], axis=0)
    cz = jnp.dot(bzl[...], mz, preferred_element_type=jnp.float32) + bzlin[...]
    ch = jnp.dot(bhl[...], mh, preferred_element_type=jnp.float32) + bhlin[...]
    bcat[...] = jnp.concatenate([cz, ch], axis=1)
    a = attn[...]
    e = jnp.exp(a - jnp.max(a))
    probs[...] = e / jnp.sum(e)


_fold = pl.pallas_call(
    _fold_body,
    out_shape=[
        jax.ShapeDtypeStruct((2 * F_IN, 2 * F_OUT), jnp.float32),
        jax.ShapeDtypeStruct((1, 2 * F_OUT), jnp.float32),
        jax.ShapeDtypeStruct((1, PERIODS), jnp.float32),
    ],
)


NB = 10
BN = N // NB


def _main_body(agg, xt, deg, wcat, bcat, probs, w1, b1, w2, b2,
               out, hacc):
    t = pl.program_id(1)
    d = jnp.maximum(deg[:, :1], 1.0)
    mean = jnp.concatenate([agg[0, i] for i in range(NHALF)], axis=1) / d
    xcat = jnp.concatenate([xt[0, i] for i in range(NHALF)], axis=1)
    cat = jnp.concatenate([mean, xcat], axis=1)
    u = jnp.dot(cat, wcat[...], preferred_element_type=jnp.float32) + bcat[...]
    z = jax.nn.sigmoid(u[:, :F_OUT])
    ht = jnp.tanh(u[:, F_OUT:])
    contrib = probs[0, t] * (1.0 - z) * ht

    @pl.when(t == 0)
    def _():
        hacc[...] = contrib

    @pl.when(t > 0)
    def _():
        hacc[...] = hacc[...] + contrib

    @pl.when(t == PERIODS - 1)
    def _():
        h = jnp.maximum(hacc[...], 0.0)
        h1 = jnp.maximum(
            jnp.dot(h, w1[...], preferred_element_type=jnp.float32) + b1[...],
            0.0)
        out[...] = jnp.dot(h1, w2[...], preferred_element_type=jnp.float32) \
            + b2[...]


_main = pl.pallas_call(
    _main_body,
    grid=(NB, PERIODS),
    in_specs=[
        pl.BlockSpec((1, NHALF, BN, FC), lambda nb, t: (t, 0, nb, 0)),
        pl.BlockSpec((1, NHALF, BN, FC), lambda nb, t: (t, 0, nb, 0)),
        pl.BlockSpec((BN, DEGW), lambda nb, t: (nb, 0)),
        pl.BlockSpec((2 * F_IN, 2 * F_OUT), lambda nb, t: (0, 0)),
        pl.BlockSpec((1, 2 * F_OUT), lambda nb, t: (0, 0)),
        pl.BlockSpec(memory_space=pltpu.SMEM),
        pl.BlockSpec((F_OUT, HID), lambda nb, t: (0, 0)),
        pl.BlockSpec((1, HID), lambda nb, t: (0, 0)),
        pl.BlockSpec((HID, OUT_DIM), lambda nb, t: (0, 0)),
        pl.BlockSpec((1, OUT_DIM), lambda nb, t: (0, 0)),
    ],
    out_specs=[
        pl.BlockSpec((BN, OUT_DIM), lambda nb, t: (nb, 0)),
        pl.BlockSpec((BN, F_OUT), lambda nb, t: (nb, 0)),
    ],
    out_shape=[
        jax.ShapeDtypeStruct((N, OUT_DIM), jnp.float32),
        jax.ShapeDtypeStruct((N, F_OUT), jnp.float32),
    ],
)


def kernel(x, edge_index, edge_attr, params):
    p = params
    xq = (x.transpose(2, 1, 0)
          .reshape(PERIODS, NHALF, FC, N)
          .transpose(0, 1, 3, 2))
    xf = xq.reshape(CHUNKS * N, FC)
    src3 = edge_index[0].reshape(NTILES, BPT, EBLK)
    dst3 = edge_index[1].reshape(NTILES, BPT, EBLK)
    zeros_in = jnp.zeros((ZROWS, FC), jnp.float32)
    ones_in = jnp.ones((EBLK, DEGW), jnp.float32)
    zcol = jnp.zeros((ROWS, DEGW), jnp.float32)

    agg, deg = _get_sc_segsum()(src3, dst3, xf, zeros_in, ones_in, zcol)
    agg4 = agg.reshape(PERIODS, NHALF, N_PAD, FC)

    wcat, bcat, probs = _fold(
        p['Wz_l'], p['Wz_r'], p['Wz_lin'],
        p['bz_l'].reshape(1, F_OUT), p['bz_lin'].reshape(1, F_OUT),
        p['Wh_l'], p['Wh_r'], p['Wh_lin'],
        p['bh_l'].reshape(1, F_OUT), p['bh_lin'].reshape(1, F_OUT),
        p['attn'].reshape(1, PERIODS))

    out, hacc = _main(agg4, xq, deg, wcat, bcat, probs,
                      p['W1'], p['b1'].reshape(1, HID),
                      p['W2'], p['b2'].reshape(1, OUT_DIM))
    return (out, hacc)
